# baseline (speedup 1.0000x reference)
import jax
import jax.numpy as jnp
from jax import lax
from jax.experimental import pallas as pl
from jax.experimental.pallas import tpu as pltpu
from jax.experimental.pallas import tpu_sc as plsc

_N = 10000
_D = 128
_E = 320000
_MLP = 256
_NUM_HH = 4852
_O_HH = 32
_O_ETH = 5
_O_REL = 9

_NC = 2
_NS = 16
_NW = _NC * _NS
_EPW = _E // _NW
_C = 128
_EPP = 10240
_NCHUNK = _EPP // _C
_G = 16
_NG = _NCHUNK // _G
_PAD_DST = 10000
_NPAD = 10240
_RPT = _NPAD // _NS
_DEGW = 16

_BR = 1000
_HH_PAD = 4864


def _make_sc_agg(with_gather):
    mesh = plsc.VectorSubcoreMesh(core_axis_name="c", subcore_axis_name="s",
                                  num_cores=_NC, num_subcores=_NS)
    out_type = [jax.ShapeDtypeStruct((_NC, _NPAD, _D), jnp.float32)]
    scratch = [
        pltpu.VMEM((2, _G, _C), jnp.int32),
        pltpu.VMEM((2, _G, _C), jnp.int32),
        pltpu.VMEM((2, _C, _D), jnp.float32),
        pltpu.VMEM_SHARED((_NPAD, _D), jnp.float32),
        pltpu.SemaphoreType.DMA,
        pltpu.SemaphoreType.DMA,
        pltpu.SemaphoreType.DMA,
    ]

    def body(y_hbm, src_hbm, dst_hbm, out_hbm, src_t, dst_t, rows_v,
             acc_sh, sem0, sem1, semi):
        c = lax.axis_index("c")
        s = lax.axis_index("s")
        wid = c * _NS + s
        sems = (sem0, sem1)

        def start_idx(g):
            gb = g % 2
            pltpu.async_copy(dst_hbm.at[wid, g], dst_t.at[gb], semi)
            if with_gather:
                pltpu.async_copy(src_hbm.at[wid, g], src_t.at[gb], semi)

        def wait_idx(g):
            gb = g % 2
            pltpu.make_async_copy(dst_hbm.at[wid, g], dst_t.at[gb],
                                  semi).wait()
            if with_gather:
                pltpu.make_async_copy(src_hbm.at[wid, g], src_t.at[gb],
                                      semi).wait()

        start_idx(0)

        def fill_rows(i, carry):
            for j in range(_D // 16):
                rows_v[0, i, pl.ds(j * 16, 16)] = carry
                rows_v[1, i, pl.ds(j * 16, 16)] = carry
            return carry

        lax.fori_loop(0, _C, fill_rows, jnp.zeros((16,), jnp.float32))
        base_r = s * _RPT
        for k in range(_RPT // _C):
            pltpu.sync_copy(rows_v.at[0],
                            acc_sh.at[pl.ds(base_r + k * _C, _C)])
        if not with_gather:
            lax.fori_loop(0, _C, fill_rows, jnp.ones((16,), jnp.float32))

        wait_idx(0)
        plsc.subcore_barrier()

        def start_gather(gb, k, b):
            if with_gather:
                pltpu.async_copy(y_hbm.at[src_t.at[gb, k]], rows_v.at[b],
                                 sems[b])

        def drain_and_scatter(gb, k, b):
            if with_gather:
                pltpu.make_async_copy(y_hbm.at[src_t.at[gb, k]],
                                      rows_v.at[b], sems[b]).wait()
            pltpu.sync_copy(rows_v.at[b], acc_sh.at[dst_t.at[gb, k]],
                            add=True)

        for g in range(_NG):
            gb = g % 2
            if g + 1 < _NG:
                start_idx(g + 1)
            start_gather(gb, 0, 0)

            def step(kk, carry):
                k0 = 2 * kk
                start_gather(gb, k0 + 1, 1)
                drain_and_scatter(gb, k0, 0)
                start_gather(gb, k0 + 2, 0)
                drain_and_scatter(gb, k0 + 1, 1)
                return carry

            lax.fori_loop(0, (_G - 1) // 2, step, 0)
            if _G % 2:
                drain_and_scatter(gb, _G - 1, 0)
            else:
                start_gather(gb, _G - 1, 1)
                drain_and_scatter(gb, _G - 2, 0)
                drain_and_scatter(gb, _G - 1, 1)
            if g + 1 < _NG:
                wait_idx(g + 1)

        plsc.subcore_barrier()
        pltpu.sync_copy(acc_sh.at[pl.ds(s * _RPT, _RPT)],
                        out_hbm.at[c, pl.ds(s * _RPT, _RPT)])

    return pl.kernel(body, out_type=out_type, mesh=mesh,
                     scratch_types=scratch)


_sc_agg = _make_sc_agg(True)
_sc_deg = _make_sc_agg(False)


def _tc_a_body(p_ref, degp_ref, x_ref, wl_ref, bl_ref, wr_ref,
               z_ref, s1_ref, s2_ref):
    deg = degp_ref[0, :, 0:1] + degp_ref[1, :, 0:1]
    agg = (p_ref[0] + p_ref[1]) / jnp.maximum(deg, 1.0)
    z = (jnp.dot(agg, wl_ref[...], preferred_element_type=jnp.float32)
         + jnp.dot(x_ref[...], wr_ref[...], preferred_element_type=jnp.float32)
         + bl_ref[...])
    z_ref[...] = z

    @pl.when(pl.program_id(0) == 0)
    def _():
        s1_ref[...] = jnp.zeros_like(s1_ref)
        s2_ref[...] = jnp.zeros_like(s2_ref)

    s1_ref[...] += jnp.sum(z, axis=0, keepdims=True)
    s2_ref[...] += jnp.sum(z * z, axis=0, keepdims=True)


def _tc_a(p, degp, x, Wl, bl, Wr):
    return pl.pallas_call(
        _tc_a_body,
        grid=(_N // _BR,),
        in_specs=[
            pl.BlockSpec((2, _BR, _D), lambda i: (0, i, 0)),
            pl.BlockSpec((2, _BR, _D), lambda i: (0, i, 0)),
            pl.BlockSpec((_BR, _D), lambda i: (i, 0)),
            pl.BlockSpec((_D, _D), lambda i: (0, 0)),
            pl.BlockSpec((1, _D), lambda i: (0, 0)),
            pl.BlockSpec((_D, _D), lambda i: (0, 0)),
        ],
        out_specs=[
            pl.BlockSpec((_BR, _D), lambda i: (i, 0)),
            pl.BlockSpec((1, _D), lambda i: (0, 0)),
            pl.BlockSpec((1, _D), lambda i: (0, 0)),
        ],
        out_shape=[
            jax.ShapeDtypeStruct((_N, _D), jnp.float32),
            jax.ShapeDtypeStruct((1, _D), jnp.float32),
            jax.ShapeDtypeStruct((1, _D), jnp.float32),
        ],
    )(p, degp, x, Wl, bl.reshape(1, _D), Wr)


def _tc_b_body(z_ref, s1_ref, s2_ref, w_ref, b_ref, a_ref, h_ref):
    mean = s1_ref[...] * (1.0 / _N)
    a = a_ref[...]
    var = s2_ref[...] * (1.0 / _N) - (2.0 * a - a * a) * mean * mean
    out = z_ref[...] - a * mean
    h = w_ref[...] * out * lax.rsqrt(var + 1e-5) + b_ref[...]
    h_ref[...] = jnp.maximum(h, 0.0)


def _tc_b(z, s1, s2, w, b, a):
    vec = pl.BlockSpec((1, _D), lambda i: (0, 0))
    return pl.pallas_call(
        _tc_b_body,
        grid=(_N // _BR,),
        in_specs=[pl.BlockSpec((_BR, _D), lambda i: (i, 0)),
                  vec, vec, vec, vec, vec],
        out_specs=pl.BlockSpec((_BR, _D), lambda i: (i, 0)),
        out_shape=jax.ShapeDtypeStruct((_N, _D), jnp.float32),
    )(z, s1, s2, w.reshape(1, _D), b.reshape(1, _D), a.reshape(1, _D))


def _heads_body(h_ref, w1a, b1a, w2a, b2a, w1b, b1b, w2b, b2b,
                w1c, b1c, w2c, b2c, oa, ob, oc):
    hh = h_ref[...]
    for w1, b1, w2, b2, o in ((w1a, b1a, w2a, b2a, oa),
                              (w1b, b1b, w2b, b2b, ob),
                              (w1c, b1c, w2c, b2c, oc)):
        t = jnp.maximum(
            jnp.dot(hh, w1[...], preferred_element_type=jnp.float32)
            + b1[...], 0.0)
        o[...] = (jnp.dot(t, w2[...], preferred_element_type=jnp.float32)
                  + b2[...])


def _heads(h, heads_params):
    def pad_w2(w2):
        return jnp.pad(w2, ((0, 0), (0, _D - w2.shape[1])))

    def pad_b2(b2):
        return jnp.pad(b2, (0, _D - b2.shape[0])).reshape(1, _D)

    args = [h]
    in_specs = [pl.BlockSpec((_HH_PAD, _D), lambda i: (0, 0))]
    for (w1, b1, w2, b2) in heads_params:
        args += [w1, b1.reshape(1, _MLP), pad_w2(w2), pad_b2(b2)]
        in_specs += [pl.BlockSpec((_D, _MLP), lambda i: (0, 0)),
                     pl.BlockSpec((1, _MLP), lambda i: (0, 0)),
                     pl.BlockSpec((_MLP, _D), lambda i: (0, 0)),
                     pl.BlockSpec((1, _D), lambda i: (0, 0))]
    ospec = pl.BlockSpec((_HH_PAD, _D), lambda i: (0, 0))
    oshape = jax.ShapeDtypeStruct((_HH_PAD, _D), jnp.float32)
    return pl.pallas_call(
        _heads_body,
        grid=(1,),
        in_specs=in_specs,
        out_specs=[ospec, ospec, ospec],
        out_shape=[oshape, oshape, oshape],
    )(*args)


def kernel(x, edge_index, Wl1, bl1, Wr1, Wl2, bl2, Wr2, Wl3, bl3, Wr3,
           gn1_w, gn1_b, gn1_a, gn2_w, gn2_b, gn2_a, gn3_w, gn3_b, gn3_a,
           hh_W1, hh_b1, hh_W2, hh_b2, eth_W1, eth_b1, eth_W2, eth_b2,
           rel_W1, rel_b1, rel_W2, rel_b2):
    pad = ((0, 0), (0, _EPP - _EPW))
    src = jnp.pad(edge_index[0].reshape(_NW, _EPW), pad,
                  constant_values=0).reshape(_NW, _NG, _G, _C)
    dst = jnp.pad(edge_index[1].reshape(_NW, _EPW), pad,
                  constant_values=_PAD_DST).reshape(_NW, _NG, _G, _C)

    (degp,) = _sc_deg(x, src, dst)
    (p1,) = _sc_agg(x, src, dst)
    z1, s11, s21 = _tc_a(p1, degp, x, Wl1, bl1, Wr1)
    h1 = _tc_b(z1, s11, s21, gn1_w, gn1_b, gn1_a)

    (p2,) = _sc_agg(h1, src, dst)
    z2, s12, s22 = _tc_a(p2, degp, h1, Wl2, bl2, Wr2)
    h2 = _tc_b(z2, s12, s22, gn2_w, gn2_b, gn2_a)

    (p3,) = _sc_agg(h2, src, dst)
    z3, s13, s23 = _tc_a(p3, degp, h2, Wl3, bl3, Wr3)
    h3 = _tc_b(z3, s13, s23, gn3_w, gn3_b, gn3_a)

    oa, ob, oc = _heads(h3, ((hh_W1, hh_b1, hh_W2, hh_b2),
                             (eth_W1, eth_b1, eth_W2, eth_b2),
                             (rel_W1, rel_b1, rel_W2, rel_b2)))
    return (oa[:_NUM_HH, :_O_HH], ob[:_NUM_HH, :_O_ETH], oc[:_NUM_HH, :_O_REL])


# speedup vs baseline: 1.0015x; 1.0015x over previous
import jax
import jax.numpy as jnp
from jax import lax
from jax.experimental import pallas as pl
from jax.experimental.pallas import tpu as pltpu
from jax.experimental.pallas import tpu_sc as plsc

_N = 10000
_D = 128
_E = 320000
_MLP = 256
_NUM_HH = 4852
_O_HH = 32
_O_ETH = 5
_O_REL = 9

_NC = 2
_NS = 16
_NW = _NC * _NS
_EPW = _E // _NW
_C = 128
_EPP = 10240
_NCHUNK = _EPP // _C
_G = 16
_NG = _NCHUNK // _G
_PAD_DST = 10000
_NPAD = 10240
_RPT = _NPAD // _NS
_DEGW = 16

_BR = 1000
_HH_PAD = 4864


def _make_sc_agg(with_gather):
    mesh = plsc.VectorSubcoreMesh(core_axis_name="c", subcore_axis_name="s",
                                  num_cores=_NC, num_subcores=_NS)
    out_type = [jax.ShapeDtypeStruct((_NC, _NPAD, _D), jnp.float32)]
    scratch = [
        pltpu.VMEM((2, _G, _C), jnp.int32),
        pltpu.VMEM((2, _G, _C), jnp.int32),
        pltpu.VMEM((2, _C, _D), jnp.float32),
        pltpu.VMEM_SHARED((_NPAD, _D), jnp.float32),
        pltpu.SemaphoreType.DMA,
        pltpu.SemaphoreType.DMA,
        pltpu.SemaphoreType.DMA,
    ]

    def body(y_hbm, src_hbm, dst_hbm, out_hbm, src_t, dst_t, rows_v,
             acc_sh, sem0, sem1, semi):
        c = lax.axis_index("c")
        s = lax.axis_index("s")
        wid = c * _NS + s
        sems = (sem0, sem1)

        def start_idx(g):
            gb = g % 2
            pltpu.async_copy(dst_hbm.at[wid, g], dst_t.at[gb], semi)
            if with_gather:
                pltpu.async_copy(src_hbm.at[wid, g], src_t.at[gb], semi)

        def wait_idx(g):
            gb = g % 2
            pltpu.make_async_copy(dst_hbm.at[wid, g], dst_t.at[gb],
                                  semi).wait()
            if with_gather:
                pltpu.make_async_copy(src_hbm.at[wid, g], src_t.at[gb],
                                      semi).wait()

        start_idx(0)

        def fill_rows(i, carry):
            for j in range(_D // 16):
                rows_v[0, i, pl.ds(j * 16, 16)] = carry
                rows_v[1, i, pl.ds(j * 16, 16)] = carry
            return carry

        lax.fori_loop(0, _C, fill_rows, jnp.zeros((16,), jnp.float32))
        base_r = s * _RPT
        for k in range(_RPT // _C):
            pltpu.sync_copy(rows_v.at[0],
                            acc_sh.at[pl.ds(base_r + k * _C, _C)])
        if not with_gather:
            lax.fori_loop(0, _C, fill_rows, jnp.ones((16,), jnp.float32))

        wait_idx(0)
        plsc.subcore_barrier()

        def start_gather(gb, k, b):
            if with_gather:
                pltpu.async_copy(y_hbm.at[src_t.at[gb, k]], rows_v.at[b],
                                 sems[b])

        def drain_and_scatter(gb, k, b):
            if with_gather:
                pltpu.make_async_copy(y_hbm.at[src_t.at[gb, k]],
                                      rows_v.at[b], sems[b]).wait()
            pltpu.sync_copy(rows_v.at[b], acc_sh.at[dst_t.at[gb, k]],
                            add=True)

        for g in range(_NG):
            gb = g % 2
            if g + 1 < _NG:
                start_idx(g + 1)
            start_gather(gb, 0, 0)

            def step(kk, carry):
                k0 = 2 * kk
                start_gather(gb, k0 + 1, 1)
                drain_and_scatter(gb, k0, 0)
                start_gather(gb, k0 + 2, 0)
                drain_and_scatter(gb, k0 + 1, 1)
                return carry

            lax.fori_loop(0, (_G - 1) // 2, step, 0)
            if _G % 2:
                drain_and_scatter(gb, _G - 1, 0)
            else:
                start_gather(gb, _G - 1, 1)
                drain_and_scatter(gb, _G - 2, 0)
                drain_and_scatter(gb, _G - 1, 1)
            if g + 1 < _NG:
                wait_idx(g + 1)

        plsc.subcore_barrier()
        pltpu.sync_copy(acc_sh.at[pl.ds(s * _RPT, _RPT)],
                        out_hbm.at[c, pl.ds(s * _RPT, _RPT)])

    return pl.kernel(body, out_type=out_type, mesh=mesh,
                     scratch_types=scratch)


_sc_agg = _make_sc_agg(True)
_sc_deg = _make_sc_agg(False)


def _tc_a_body(p_ref, degp_ref, x_ref, wl_ref, bl_ref, wr_ref,
               z_ref, s1_ref, s2_ref):
    deg = degp_ref[0, :, 0:1] + degp_ref[1, :, 0:1]
    agg = (p_ref[0] + p_ref[1]) / jnp.maximum(deg, 1.0)
    z = (jnp.dot(agg, wl_ref[...], preferred_element_type=jnp.float32)
         + jnp.dot(x_ref[...], wr_ref[...], preferred_element_type=jnp.float32)
         + bl_ref[...])
    z_ref[...] = z

    @pl.when(pl.program_id(0) == 0)
    def _():
        s1_ref[...] = jnp.zeros_like(s1_ref)
        s2_ref[...] = jnp.zeros_like(s2_ref)

    s1_ref[...] += jnp.sum(z, axis=0, keepdims=True)
    s2_ref[...] += jnp.sum(z * z, axis=0, keepdims=True)


def _tc_a(p, degp, x, Wl, bl, Wr):
    return pl.pallas_call(
        _tc_a_body,
        grid=(_N // _BR,),
        in_specs=[
            pl.BlockSpec((2, _BR, _D), lambda i: (0, i, 0)),
            pl.BlockSpec((2, _BR, _D), lambda i: (0, i, 0)),
            pl.BlockSpec((_BR, _D), lambda i: (i, 0)),
            pl.BlockSpec((_D, _D), lambda i: (0, 0)),
            pl.BlockSpec((1, _D), lambda i: (0, 0)),
            pl.BlockSpec((_D, _D), lambda i: (0, 0)),
        ],
        out_specs=[
            pl.BlockSpec((_BR, _D), lambda i: (i, 0)),
            pl.BlockSpec((1, _D), lambda i: (0, 0)),
            pl.BlockSpec((1, _D), lambda i: (0, 0)),
        ],
        out_shape=[
            jax.ShapeDtypeStruct((_N, _D), jnp.float32),
            jax.ShapeDtypeStruct((1, _D), jnp.float32),
            jax.ShapeDtypeStruct((1, _D), jnp.float32),
        ],
    )(p, degp, x, Wl, bl.reshape(1, _D), Wr)


def _tc_b_body(z_ref, s1_ref, s2_ref, w_ref, b_ref, a_ref, h_ref):
    mean = s1_ref[...] * (1.0 / _N)
    a = a_ref[...]
    var = s2_ref[...] * (1.0 / _N) - (2.0 * a - a * a) * mean * mean
    out = z_ref[...] - a * mean
    h = w_ref[...] * out * lax.rsqrt(var + 1e-5) + b_ref[...]
    h_ref[...] = jnp.maximum(h, 0.0)


def _tc_b(z, s1, s2, w, b, a):
    vec = pl.BlockSpec((1, _D), lambda i: (0, 0))
    return pl.pallas_call(
        _tc_b_body,
        grid=(_N // _BR,),
        in_specs=[pl.BlockSpec((_BR, _D), lambda i: (i, 0)),
                  vec, vec, vec, vec, vec],
        out_specs=pl.BlockSpec((_BR, _D), lambda i: (i, 0)),
        out_shape=jax.ShapeDtypeStruct((_N, _D), jnp.float32),
    )(z, s1, s2, w.reshape(1, _D), b.reshape(1, _D), a.reshape(1, _D))


def _heads_body(h_ref, w1a, b1a, w2a, b2a, w1b, b1b, w2b, b2b,
                w1c, b1c, w2c, b2c, oa, ob, oc):
    hh = h_ref[...]
    for w1, b1, w2, b2, o in ((w1a, b1a, w2a, b2a, oa),
                              (w1b, b1b, w2b, b2b, ob),
                              (w1c, b1c, w2c, b2c, oc)):
        t = jnp.maximum(
            jnp.dot(hh, w1[...], preferred_element_type=jnp.float32)
            + b1[...], 0.0)
        o[...] = (jnp.dot(t, w2[...], preferred_element_type=jnp.float32)
                  + b2[...])


def _heads(h, heads_params):
    def pad_w2(w2):
        return jnp.pad(w2, ((0, 0), (0, _D - w2.shape[1])))

    def pad_b2(b2):
        return jnp.pad(b2, (0, _D - b2.shape[0])).reshape(1, _D)

    args = [h]
    in_specs = [pl.BlockSpec((_HH_PAD, _D), lambda i: (0, 0))]
    for (w1, b1, w2, b2) in heads_params:
        args += [w1, b1.reshape(1, _MLP), pad_w2(w2), pad_b2(b2)]
        in_specs += [pl.BlockSpec((_D, _MLP), lambda i: (0, 0)),
                     pl.BlockSpec((1, _MLP), lambda i: (0, 0)),
                     pl.BlockSpec((_MLP, _D), lambda i: (0, 0)),
                     pl.BlockSpec((1, _D), lambda i: (0, 0))]
    ospec = pl.BlockSpec((_HH_PAD, _D), lambda i: (0, 0))
    oshape = jax.ShapeDtypeStruct((_HH_PAD, _D), jnp.float32)
    return pl.pallas_call(
        _heads_body,
        grid=(1,),
        in_specs=in_specs,
        out_specs=[ospec, ospec, ospec],
        out_shape=[oshape, oshape, oshape],
    )(*args)


def kernel(x, edge_index, Wl1, bl1, Wr1, Wl2, bl2, Wr2, Wl3, bl3, Wr3,
           gn1_w, gn1_b, gn1_a, gn2_w, gn2_b, gn2_a, gn3_w, gn3_b, gn3_a,
           hh_W1, hh_b1, hh_W2, hh_b2, eth_W1, eth_b1, eth_W2, eth_b2,
           rel_W1, rel_b1, rel_W2, rel_b2):
    pad = ((0, 0), (0, _EPP - _EPW))
    src = jnp.pad(edge_index[0].reshape(_NW, _EPW), pad,
                  constant_values=0).reshape(_NW, _NG, _G, _C)
    pad_rows = jnp.broadcast_to(
        _PAD_DST + jnp.arange(_EPP - _EPW, dtype=jnp.int32), (_NW, _EPP - _EPW))
    dst = jnp.concatenate(
        [edge_index[1].reshape(_NW, _EPW), pad_rows],
        axis=1).reshape(_NW, _NG, _G, _C)

    (degp,) = _sc_deg(x, src, dst)
    (p1,) = _sc_agg(x, src, dst)
    z1, s11, s21 = _tc_a(p1, degp, x, Wl1, bl1, Wr1)
    h1 = _tc_b(z1, s11, s21, gn1_w, gn1_b, gn1_a)

    (p2,) = _sc_agg(h1, src, dst)
    z2, s12, s22 = _tc_a(p2, degp, h1, Wl2, bl2, Wr2)
    h2 = _tc_b(z2, s12, s22, gn2_w, gn2_b, gn2_a)

    (p3,) = _sc_agg(h2, src, dst)
    z3, s13, s23 = _tc_a(p3, degp, h2, Wl3, bl3, Wr3)
    h3 = _tc_b(z3, s13, s23, gn3_w, gn3_b, gn3_a)

    oa, ob, oc = _heads(h3, ((hh_W1, hh_b1, hh_W2, hh_b2),
                             (eth_W1, eth_b1, eth_W2, eth_b2),
                             (rel_W1, rel_b1, rel_W2, rel_b2)))
    return (oa[:_NUM_HH, :_O_HH], ob[:_NUM_HH, :_O_ETH], oc[:_NUM_HH, :_O_REL])


# speedup vs baseline: 2.5213x; 2.5175x over previous
import jax
import jax.numpy as jnp
from jax import lax
from jax.experimental import pallas as pl
from jax.experimental.pallas import tpu as pltpu
from jax.experimental.pallas import tpu_sc as plsc

_N = 10000
_D = 128
_E = 320000
_MLP = 256
_NUM_HH = 4852
_O_HH = 32
_O_ETH = 5
_O_REL = 9

_NC = 2
_NS = 16
_NW = _NC * _NS
_EPW = _E // _NW
_C = 80
_NCHUNK = _EPW // _C
_G = 25
_NG = _NCHUNK // _G
_NPAD = 10240
_RPT = _NPAD // _NS
_DEGW = 16

_BR = 1000
_HH_PAD = 4864


def _make_sc_agg(with_gather):
    mesh = plsc.VectorSubcoreMesh(core_axis_name="c", subcore_axis_name="s",
                                  num_cores=_NC, num_subcores=_NS)
    out_type = [jax.ShapeDtypeStruct((_NC, _NPAD, _D), jnp.float32)]
    scratch = [
        pltpu.VMEM((2, _G, _C), jnp.int32),
        pltpu.VMEM((2, _G, _C), jnp.int32),
        pltpu.VMEM((2, _C, _D), jnp.float32),
        pltpu.VMEM_SHARED((_NPAD, _D), jnp.float32),
        pltpu.SemaphoreType.DMA,
        pltpu.SemaphoreType.DMA,
        pltpu.SemaphoreType.DMA,
    ]

    def body(y_hbm, src_hbm, dst_hbm, out_hbm, src_t, dst_t, rows_v,
             acc_sh, sem0, sem1, semi):
        c = lax.axis_index("c")
        s = lax.axis_index("s")
        wid = c * _NS + s
        sems = (sem0, sem1)

        def start_idx(g):
            gb = g % 2
            pltpu.async_copy(dst_hbm.at[wid, g], dst_t.at[gb], semi)
            if with_gather:
                pltpu.async_copy(src_hbm.at[wid, g], src_t.at[gb], semi)

        def wait_idx(g):
            gb = g % 2
            pltpu.make_async_copy(dst_hbm.at[wid, g], dst_t.at[gb],
                                  semi).wait()
            if with_gather:
                pltpu.make_async_copy(src_hbm.at[wid, g], src_t.at[gb],
                                      semi).wait()

        start_idx(0)

        def fill_rows(i, carry):
            for j in range(_D // 16):
                rows_v[0, i, pl.ds(j * 16, 16)] = carry
                rows_v[1, i, pl.ds(j * 16, 16)] = carry
            return carry

        lax.fori_loop(0, _C, fill_rows, jnp.zeros((16,), jnp.float32))
        base_r = s * _RPT
        for k in range(_RPT // _C):
            pltpu.sync_copy(rows_v.at[0],
                            acc_sh.at[pl.ds(base_r + k * _C, _C)])
        if not with_gather:
            lax.fori_loop(0, _C, fill_rows, jnp.ones((16,), jnp.float32))

        wait_idx(0)
        plsc.subcore_barrier()

        def start_gather(gb, k, b):
            if with_gather:
                pltpu.async_copy(y_hbm.at[src_t.at[gb, k]], rows_v.at[b],
                                 sems[b])

        def drain_and_scatter(gb, k, b):
            if with_gather:
                pltpu.make_async_copy(y_hbm.at[src_t.at[gb, k]],
                                      rows_v.at[b], sems[b]).wait()
            pltpu.sync_copy(rows_v.at[b], acc_sh.at[dst_t.at[gb, k]],
                            add=True)

        for g in range(_NG):
            gb = g % 2
            if g + 1 < _NG:
                start_idx(g + 1)
            start_gather(gb, 0, 0)

            def step(kk, carry):
                k0 = 2 * kk
                start_gather(gb, k0 + 1, 1)
                drain_and_scatter(gb, k0, 0)
                start_gather(gb, k0 + 2, 0)
                drain_and_scatter(gb, k0 + 1, 1)
                return carry

            lax.fori_loop(0, (_G - 1) // 2, step, 0)
            if _G % 2:
                drain_and_scatter(gb, _G - 1, 0)
            else:
                start_gather(gb, _G - 1, 1)
                drain_and_scatter(gb, _G - 2, 0)
                drain_and_scatter(gb, _G - 1, 1)
            if g + 1 < _NG:
                wait_idx(g + 1)

        plsc.subcore_barrier()
        pltpu.sync_copy(acc_sh.at[pl.ds(s * _RPT, _RPT)],
                        out_hbm.at[c, pl.ds(s * _RPT, _RPT)])

    return pl.kernel(body, out_type=out_type, mesh=mesh,
                     scratch_types=scratch)


_sc_agg = _make_sc_agg(True)
_sc_deg = _make_sc_agg(False)


def _tc_a_body(p_ref, degp_ref, x_ref, wl_ref, bl_ref, wr_ref,
               z_ref, s1_ref, s2_ref):
    deg = degp_ref[0, :, 0:1] + degp_ref[1, :, 0:1]
    agg = (p_ref[0] + p_ref[1]) / jnp.maximum(deg, 1.0)
    z = (jnp.dot(agg, wl_ref[...], preferred_element_type=jnp.float32)
         + jnp.dot(x_ref[...], wr_ref[...], preferred_element_type=jnp.float32)
         + bl_ref[...])
    z_ref[...] = z

    @pl.when(pl.program_id(0) == 0)
    def _():
        s1_ref[...] = jnp.zeros_like(s1_ref)
        s2_ref[...] = jnp.zeros_like(s2_ref)

    s1_ref[...] += jnp.sum(z, axis=0, keepdims=True)
    s2_ref[...] += jnp.sum(z * z, axis=0, keepdims=True)


def _tc_a(p, degp, x, Wl, bl, Wr):
    return pl.pallas_call(
        _tc_a_body,
        grid=(_N // _BR,),
        in_specs=[
            pl.BlockSpec((2, _BR, _D), lambda i: (0, i, 0)),
            pl.BlockSpec((2, _BR, _D), lambda i: (0, i, 0)),
            pl.BlockSpec((_BR, _D), lambda i: (i, 0)),
            pl.BlockSpec((_D, _D), lambda i: (0, 0)),
            pl.BlockSpec((1, _D), lambda i: (0, 0)),
            pl.BlockSpec((_D, _D), lambda i: (0, 0)),
        ],
        out_specs=[
            pl.BlockSpec((_BR, _D), lambda i: (i, 0)),
            pl.BlockSpec((1, _D), lambda i: (0, 0)),
            pl.BlockSpec((1, _D), lambda i: (0, 0)),
        ],
        out_shape=[
            jax.ShapeDtypeStruct((_N, _D), jnp.float32),
            jax.ShapeDtypeStruct((1, _D), jnp.float32),
            jax.ShapeDtypeStruct((1, _D), jnp.float32),
        ],
    )(p, degp, x, Wl, bl.reshape(1, _D), Wr)


def _tc_b_body(z_ref, s1_ref, s2_ref, w_ref, b_ref, a_ref, h_ref):
    mean = s1_ref[...] * (1.0 / _N)
    a = a_ref[...]
    var = s2_ref[...] * (1.0 / _N) - (2.0 * a - a * a) * mean * mean
    out = z_ref[...] - a * mean
    h = w_ref[...] * out * lax.rsqrt(var + 1e-5) + b_ref[...]
    h_ref[...] = jnp.maximum(h, 0.0)


def _tc_b(z, s1, s2, w, b, a):
    vec = pl.BlockSpec((1, _D), lambda i: (0, 0))
    return pl.pallas_call(
        _tc_b_body,
        grid=(_N // _BR,),
        in_specs=[pl.BlockSpec((_BR, _D), lambda i: (i, 0)),
                  vec, vec, vec, vec, vec],
        out_specs=pl.BlockSpec((_BR, _D), lambda i: (i, 0)),
        out_shape=jax.ShapeDtypeStruct((_N, _D), jnp.float32),
    )(z, s1, s2, w.reshape(1, _D), b.reshape(1, _D), a.reshape(1, _D))


def _heads_body(h_ref, w1a, b1a, w2a, b2a, w1b, b1b, w2b, b2b,
                w1c, b1c, w2c, b2c, oa, ob, oc):
    hh = h_ref[...]
    for w1, b1, w2, b2, o in ((w1a, b1a, w2a, b2a, oa),
                              (w1b, b1b, w2b, b2b, ob),
                              (w1c, b1c, w2c, b2c, oc)):
        t = jnp.maximum(
            jnp.dot(hh, w1[...], preferred_element_type=jnp.float32)
            + b1[...], 0.0)
        o[...] = (jnp.dot(t, w2[...], preferred_element_type=jnp.float32)
                  + b2[...])


def _heads(h, heads_params):
    def pad_w2(w2):
        return jnp.pad(w2, ((0, 0), (0, _D - w2.shape[1])))

    def pad_b2(b2):
        return jnp.pad(b2, (0, _D - b2.shape[0])).reshape(1, _D)

    args = [h]
    in_specs = [pl.BlockSpec((_HH_PAD, _D), lambda i: (0, 0))]
    for (w1, b1, w2, b2) in heads_params:
        args += [w1, b1.reshape(1, _MLP), pad_w2(w2), pad_b2(b2)]
        in_specs += [pl.BlockSpec((_D, _MLP), lambda i: (0, 0)),
                     pl.BlockSpec((1, _MLP), lambda i: (0, 0)),
                     pl.BlockSpec((_MLP, _D), lambda i: (0, 0)),
                     pl.BlockSpec((1, _D), lambda i: (0, 0))]
    ospec = pl.BlockSpec((_HH_PAD, _D), lambda i: (0, 0))
    oshape = jax.ShapeDtypeStruct((_HH_PAD, _D), jnp.float32)
    return pl.pallas_call(
        _heads_body,
        grid=(1,),
        in_specs=in_specs,
        out_specs=[ospec, ospec, ospec],
        out_shape=[oshape, oshape, oshape],
    )(*args)


def kernel(x, edge_index, Wl1, bl1, Wr1, Wl2, bl2, Wr2, Wl3, bl3, Wr3,
           gn1_w, gn1_b, gn1_a, gn2_w, gn2_b, gn2_a, gn3_w, gn3_b, gn3_a,
           hh_W1, hh_b1, hh_W2, hh_b2, eth_W1, eth_b1, eth_W2, eth_b2,
           rel_W1, rel_b1, rel_W2, rel_b2):
    src = edge_index[0].reshape(_NW, _NG, _G, _C)
    dst = edge_index[1].reshape(_NW, _NG, _G, _C)

    (degp,) = _sc_deg(x, src, dst)
    (p1,) = _sc_agg(x, src, dst)
    z1, s11, s21 = _tc_a(p1, degp, x, Wl1, bl1, Wr1)
    h1 = _tc_b(z1, s11, s21, gn1_w, gn1_b, gn1_a)

    (p2,) = _sc_agg(h1, src, dst)
    z2, s12, s22 = _tc_a(p2, degp, h1, Wl2, bl2, Wr2)
    h2 = _tc_b(z2, s12, s22, gn2_w, gn2_b, gn2_a)

    (p3,) = _sc_agg(h2, src, dst)
    z3, s13, s23 = _tc_a(p3, degp, h2, Wl3, bl3, Wr3)
    h3 = _tc_b(z3, s13, s23, gn3_w, gn3_b, gn3_a)

    oa, ob, oc = _heads(h3, ((hh_W1, hh_b1, hh_W2, hh_b2),
                             (eth_W1, eth_b1, eth_W2, eth_b2),
                             (rel_W1, rel_b1, rel_W2, rel_b2)))
    return (oa[:_NUM_HH, :_O_HH], ob[:_NUM_HH, :_O_ETH], oc[:_NUM_HH, :_O_REL])


# speedup vs baseline: 2.5849x; 1.0252x over previous
import jax
import jax.numpy as jnp
from jax import lax
from jax.experimental import pallas as pl
from jax.experimental.pallas import tpu as pltpu
from jax.experimental.pallas import tpu_sc as plsc

_N = 10000
_D = 128
_E = 320000
_MLP = 256
_NUM_HH = 4852
_O_HH = 32
_O_ETH = 5
_O_REL = 9

_NC = 2
_NS = 16
_NW = _NC * _NS
_EPW = _E // _NW
_C = 80
_NCHUNK = _EPW // _C
_G = 25
_NG = _NCHUNK // _G
_NPAD = 10240
_RPT = _NPAD // _NS
_DEGW = 16

_BR = 1000
_HH_PAD = 4864


def _make_sc_agg(with_gather):
    mesh = plsc.VectorSubcoreMesh(core_axis_name="c", subcore_axis_name="s",
                                  num_cores=_NC, num_subcores=_NS)
    out_type = [jax.ShapeDtypeStruct((_NC, _NPAD, _D), jnp.float32)]
    scratch = [
        pltpu.VMEM((2, _G, _C), jnp.int32),
        pltpu.VMEM((2, _G, _C), jnp.int32),
        pltpu.VMEM((2, _C, _D), jnp.float32),
        pltpu.VMEM_SHARED((_NPAD, _D), jnp.float32),
        pltpu.SemaphoreType.DMA,
        pltpu.SemaphoreType.DMA,
        pltpu.SemaphoreType.DMA,
    ]

    def body(y_hbm, src_hbm, dst_hbm, out_hbm, src_t, dst_t, rows_v,
             acc_sh, sem0, sem1, semi):
        c = lax.axis_index("c")
        s = lax.axis_index("s")
        wid = c * _NS + s
        sems = (sem0, sem1)

        def start_idx(g):
            gb = g % 2
            pltpu.async_copy(dst_hbm.at[wid, g], dst_t.at[gb], semi)
            if with_gather:
                pltpu.async_copy(src_hbm.at[wid, g], src_t.at[gb], semi)

        def wait_idx(g):
            gb = g % 2
            pltpu.make_async_copy(dst_hbm.at[wid, g], dst_t.at[gb],
                                  semi).wait()
            if with_gather:
                pltpu.make_async_copy(src_hbm.at[wid, g], src_t.at[gb],
                                      semi).wait()

        start_idx(0)

        def fill_rows(i, carry):
            for j in range(_D // 16):
                rows_v[0, i, pl.ds(j * 16, 16)] = carry
                rows_v[1, i, pl.ds(j * 16, 16)] = carry
            return carry

        lax.fori_loop(0, _C, fill_rows, jnp.zeros((16,), jnp.float32))
        base_r = s * _RPT
        for k in range(_RPT // _C):
            pltpu.sync_copy(rows_v.at[0],
                            acc_sh.at[pl.ds(base_r + k * _C, _C)])
        if not with_gather:
            lax.fori_loop(0, _C, fill_rows, jnp.ones((16,), jnp.float32))

        wait_idx(0)
        plsc.subcore_barrier()

        def start_gather(gb, k, b):
            if with_gather:
                pltpu.async_copy(y_hbm.at[src_t.at[gb, k]], rows_v.at[b],
                                 sems[b])

        def drain_and_scatter(gb, k, b):
            if with_gather:
                pltpu.make_async_copy(y_hbm.at[src_t.at[gb, k]],
                                      rows_v.at[b], sems[b]).wait()
            pltpu.sync_copy(rows_v.at[b], acc_sh.at[dst_t.at[gb, k]],
                            add=True)

        for g in range(_NG):
            gb = g % 2
            if g + 1 < _NG:
                start_idx(g + 1)
            start_gather(gb, 0, 0)

            def step(kk, carry):
                k0 = 2 * kk
                start_gather(gb, k0 + 1, 1)
                drain_and_scatter(gb, k0, 0)
                start_gather(gb, k0 + 2, 0)
                drain_and_scatter(gb, k0 + 1, 1)
                return carry

            lax.fori_loop(0, (_G - 1) // 2, step, 0)
            if _G % 2:
                drain_and_scatter(gb, _G - 1, 0)
            else:
                start_gather(gb, _G - 1, 1)
                drain_and_scatter(gb, _G - 2, 0)
                drain_and_scatter(gb, _G - 1, 1)
            if g + 1 < _NG:
                wait_idx(g + 1)

        plsc.subcore_barrier()
        pltpu.sync_copy(acc_sh.at[pl.ds(s * _RPT, _RPT)],
                        out_hbm.at[c, pl.ds(s * _RPT, _RPT)])

    return pl.kernel(body, out_type=out_type, mesh=mesh,
                     scratch_types=scratch)


_sc_agg = _make_sc_agg(True)
_sc_deg = _make_sc_agg(False)


_NBLK = _N // _BR


def _tc_layer(p, degp, x, Wl, bl, Wr, gw, gb, ga, heads_params=None):
    with_heads = heads_params is not None

    def body(p_ref, degp_ref, x_ref, wl_ref, bl_ref, wr_ref,
             gw_ref, gb_ref, ga_ref, *rest):
        if with_heads:
            hw = rest[:12]
            h_ref, oa_ref, ob_ref, oc_ref, z_scr, s1, s2 = rest[12:]
        else:
            h_ref, z_scr, s1, s2 = rest
        ph = pl.program_id(0)
        i = pl.program_id(1)

        @pl.when(ph == 0)
        def _():
            deg = degp_ref[0, :, 0:1] + degp_ref[1, :, 0:1]
            agg = (p_ref[0] + p_ref[1]) / jnp.maximum(deg, 1.0)
            z = (jnp.dot(agg, wl_ref[...],
                         preferred_element_type=jnp.float32)
                 + jnp.dot(x_ref[...], wr_ref[...],
                           preferred_element_type=jnp.float32)
                 + bl_ref[...])
            z_scr[pl.ds(i * _BR, _BR)] = z

            @pl.when(i == 0)
            def _():
                s1[...] = jnp.zeros_like(s1)
                s2[...] = jnp.zeros_like(s2)

            s1[...] += jnp.sum(z, axis=0, keepdims=True)
            s2[...] += jnp.sum(z * z, axis=0, keepdims=True)

        @pl.when(ph == 1)
        def _():
            mean = s1[...] * (1.0 / _N)
            a = ga_ref[...]
            var = s2[...] * (1.0 / _N) - (2.0 * a - a * a) * mean * mean
            z = z_scr[pl.ds(i * _BR, _BR)]
            h = (gw_ref[...] * (z - a * mean) * lax.rsqrt(var + 1e-5)
                 + gb_ref[...])
            h = jnp.maximum(h, 0.0)
            h_ref[...] = h
            if with_heads:
                @pl.when(i * _BR < _NUM_HH)
                def _():
                    for j, o_ref in enumerate((oa_ref, ob_ref, oc_ref)):
                        w1, b1, w2, b2 = hw[4 * j:4 * j + 4]
                        t = jnp.maximum(
                            jnp.dot(h, w1[...],
                                    preferred_element_type=jnp.float32)
                            + b1[...], 0.0)
                        o_ref[...] = (
                            jnp.dot(t, w2[...],
                                    preferred_element_type=jnp.float32)
                            + b2[...])

    def in3(ph, i):
        return (0, jnp.where(ph == 0, i, _NBLK - 1), 0)

    def in2(ph, i):
        return (jnp.where(ph == 0, i, _NBLK - 1), 0)

    def out2(ph, i):
        return (jnp.where(ph == 1, i, 0), 0)

    bcast = lambda ph, i: (0, 0)
    args = [p, degp, x, Wl, bl.reshape(1, _D), Wr,
            gw.reshape(1, _D), gb.reshape(1, _D), ga.reshape(1, _D)]
    in_specs = [
        pl.BlockSpec((2, _BR, _D), in3),
        pl.BlockSpec((2, _BR, _D), in3),
        pl.BlockSpec((_BR, _D), in2),
        pl.BlockSpec((_D, _D), bcast),
        pl.BlockSpec((1, _D), bcast),
        pl.BlockSpec((_D, _D), bcast),
        pl.BlockSpec((1, _D), bcast),
        pl.BlockSpec((1, _D), bcast),
        pl.BlockSpec((1, _D), bcast),
    ]
    oblk = pl.BlockSpec((_BR, _D), out2)
    osh = jax.ShapeDtypeStruct((_N, _D), jnp.float32)
    out_specs = [oblk]
    out_shape = [osh]
    if with_heads:
        for (w1, b1, w2, b2) in heads_params:
            args += [w1, b1.reshape(1, _MLP),
                     jnp.pad(w2, ((0, 0), (0, _D - w2.shape[1]))),
                     jnp.pad(b2, (0, _D - b2.shape[0])).reshape(1, _D)]
            in_specs += [pl.BlockSpec((_D, _MLP), bcast),
                         pl.BlockSpec((1, _MLP), bcast),
                         pl.BlockSpec((_MLP, _D), bcast),
                         pl.BlockSpec((1, _D), bcast)]
        out_specs += [oblk, oblk, oblk]
        out_shape += [osh, osh, osh]

    return pl.pallas_call(
        body,
        grid=(2, _NBLK),
        in_specs=in_specs,
        out_specs=out_specs,
        out_shape=out_shape,
        scratch_shapes=[pltpu.VMEM((_N, _D), jnp.float32),
                        pltpu.VMEM((1, _D), jnp.float32),
                        pltpu.VMEM((1, _D), jnp.float32)],
    )(*args)


def kernel(x, edge_index, Wl1, bl1, Wr1, Wl2, bl2, Wr2, Wl3, bl3, Wr3,
           gn1_w, gn1_b, gn1_a, gn2_w, gn2_b, gn2_a, gn3_w, gn3_b, gn3_a,
           hh_W1, hh_b1, hh_W2, hh_b2, eth_W1, eth_b1, eth_W2, eth_b2,
           rel_W1, rel_b1, rel_W2, rel_b2):
    src = edge_index[0].reshape(_NW, _NG, _G, _C)
    dst = edge_index[1].reshape(_NW, _NG, _G, _C)

    (degp,) = _sc_deg(x, src, dst)
    (p1,) = _sc_agg(x, src, dst)
    (h1,) = _tc_layer(p1, degp, x, Wl1, bl1, Wr1, gn1_w, gn1_b, gn1_a)

    (p2,) = _sc_agg(h1, src, dst)
    (h2,) = _tc_layer(p2, degp, h1, Wl2, bl2, Wr2, gn2_w, gn2_b, gn2_a)

    (p3,) = _sc_agg(h2, src, dst)
    _, oa, ob, oc = _tc_layer(
        p3, degp, h2, Wl3, bl3, Wr3, gn3_w, gn3_b, gn3_a,
        heads_params=((hh_W1, hh_b1, hh_W2, hh_b2),
                      (eth_W1, eth_b1, eth_W2, eth_b2),
                      (rel_W1, rel_b1, rel_W2, rel_b2)))
    return (oa[:_NUM_HH, :_O_HH], ob[:_NUM_HH, :_O_ETH], oc[:_NUM_HH, :_O_REL])


# speedup vs baseline: 2.8757x; 1.1125x over previous
import jax
import jax.numpy as jnp
from jax import lax
from jax.experimental import pallas as pl
from jax.experimental.pallas import tpu as pltpu
from jax.experimental.pallas import tpu_sc as plsc

_N = 10000
_D = 128
_E = 320000
_MLP = 256
_NUM_HH = 4852
_O_HH = 32
_O_ETH = 5
_O_REL = 9

_NC = 2
_NS = 16
_NW = _NC * _NS
_EPW = _E // _NW
_C = 80
_NCHUNK = _EPW // _C
_G = 25
_NG = _NCHUNK // _G
_NBUF = 3
_NPAD = 10240
_RPT = _NPAD // _NS
_DEGW = 16

_BR = 1000
_HH_PAD = 4864


def _make_sc_agg(with_gather):
    mesh = plsc.VectorSubcoreMesh(core_axis_name="c", subcore_axis_name="s",
                                  num_cores=_NC, num_subcores=_NS)
    out_type = [jax.ShapeDtypeStruct((_NC, _NPAD, _D), jnp.float32)]
    scratch = [
        pltpu.VMEM((2, _G, _C), jnp.int32),
        pltpu.VMEM((2, _G, _C), jnp.int32),
        pltpu.VMEM((_NBUF, _C, _D), jnp.float32),
        pltpu.VMEM_SHARED((_NPAD, _D), jnp.float32),
        pltpu.SemaphoreType.DMA,
        pltpu.SemaphoreType.DMA,
        pltpu.SemaphoreType.DMA,
        pltpu.SemaphoreType.DMA,
        pltpu.SemaphoreType.DMA,
    ]

    def body(y_hbm, src_hbm, dst_hbm, out_hbm, src_t, dst_t, rows_v,
             acc_sh, sem0, sem1, sem2, sem3, semi):
        c = lax.axis_index("c")
        s = lax.axis_index("s")
        wid = c * _NS + s
        sems = (sem0, sem1, sem2, sem3)

        def start_idx(g):
            gb = g % 2
            pltpu.async_copy(dst_hbm.at[wid, g], dst_t.at[gb], semi)
            if with_gather:
                pltpu.async_copy(src_hbm.at[wid, g], src_t.at[gb], semi)

        def wait_idx(g):
            gb = g % 2
            pltpu.make_async_copy(dst_hbm.at[wid, g], dst_t.at[gb],
                                  semi).wait()
            if with_gather:
                pltpu.make_async_copy(src_hbm.at[wid, g], src_t.at[gb],
                                      semi).wait()

        start_idx(0)

        def fill_rows(i, carry):
            for b in range(_NBUF):
                for j in range(_D // 16):
                    rows_v[b, i, pl.ds(j * 16, 16)] = carry
            return carry

        lax.fori_loop(0, _C, fill_rows, jnp.zeros((16,), jnp.float32))
        base_r = s * _RPT
        for k in range(_RPT // _C):
            pltpu.sync_copy(rows_v.at[0],
                            acc_sh.at[pl.ds(base_r + k * _C, _C)])
        if not with_gather:
            lax.fori_loop(0, _C, fill_rows, jnp.ones((16,), jnp.float32))

        wait_idx(0)
        plsc.subcore_barrier()

        def start_gather(gb, k, b):
            if with_gather:
                pltpu.async_copy(y_hbm.at[src_t.at[gb, k]], rows_v.at[b],
                                 sems[b])

        def drain_and_scatter(gb, k, b):
            if with_gather:
                pltpu.make_async_copy(y_hbm.at[src_t.at[gb, k]],
                                      rows_v.at[b], sems[b]).wait()
            pltpu.sync_copy(rows_v.at[b], acc_sh.at[dst_t.at[gb, k]],
                            add=True)

        steady = (_G - _NBUF - 1) // _NBUF

        for g in range(_NG):
            gb = g % 2
            if g + 1 < _NG:
                start_idx(g + 1)
            for j in range(_NBUF - 1):
                start_gather(gb, j, j)

            def step(kk, carry):
                for j in range(_NBUF):
                    k = _NBUF * kk + j
                    start_gather(gb, k + _NBUF - 1,
                                 (j + _NBUF - 1) % _NBUF)
                    drain_and_scatter(gb, k, j)
                return carry

            lax.fori_loop(0, steady, step, 0)
            for k in range(_NBUF * steady, _G):
                if k + _NBUF - 1 < _G:
                    start_gather(gb, k + _NBUF - 1,
                                 (k + _NBUF - 1) % _NBUF)
                drain_and_scatter(gb, k, k % _NBUF)
            if g + 1 < _NG:
                wait_idx(g + 1)

        plsc.subcore_barrier()
        pltpu.sync_copy(acc_sh.at[pl.ds(s * _RPT, _RPT)],
                        out_hbm.at[c, pl.ds(s * _RPT, _RPT)])

    return pl.kernel(body, out_type=out_type, mesh=mesh,
                     scratch_types=scratch)


_sc_agg = _make_sc_agg(True)
_sc_deg = _make_sc_agg(False)


_NBLK = _N // _BR


def _tc_layer(p, degp, x, Wl, bl, Wr, gw, gb, ga, heads_params=None):
    with_heads = heads_params is not None

    def body(p_ref, degp_ref, x_ref, wl_ref, bl_ref, wr_ref,
             gw_ref, gb_ref, ga_ref, *rest):
        if with_heads:
            hw = rest[:12]
            h_ref, oa_ref, ob_ref, oc_ref, z_scr, s1, s2 = rest[12:]
        else:
            h_ref, z_scr, s1, s2 = rest
        ph = pl.program_id(0)
        i = pl.program_id(1)

        @pl.when(ph == 0)
        def _():
            deg = degp_ref[0, :, 0:1] + degp_ref[1, :, 0:1]
            agg = (p_ref[0] + p_ref[1]) / jnp.maximum(deg, 1.0)
            z = (jnp.dot(agg, wl_ref[...],
                         preferred_element_type=jnp.float32)
                 + jnp.dot(x_ref[...], wr_ref[...],
                           preferred_element_type=jnp.float32)
                 + bl_ref[...])
            z_scr[pl.ds(i * _BR, _BR)] = z

            @pl.when(i == 0)
            def _():
                s1[...] = jnp.zeros_like(s1)
                s2[...] = jnp.zeros_like(s2)

            s1[...] += jnp.sum(z, axis=0, keepdims=True)
            s2[...] += jnp.sum(z * z, axis=0, keepdims=True)

        @pl.when(ph == 1)
        def _():
            mean = s1[...] * (1.0 / _N)
            a = ga_ref[...]
            var = s2[...] * (1.0 / _N) - (2.0 * a - a * a) * mean * mean
            z = z_scr[pl.ds(i * _BR, _BR)]
            h = (gw_ref[...] * (z - a * mean) * lax.rsqrt(var + 1e-5)
                 + gb_ref[...])
            h = jnp.maximum(h, 0.0)
            h_ref[...] = h
            if with_heads:
                @pl.when(i * _BR < _NUM_HH)
                def _():
                    for j, o_ref in enumerate((oa_ref, ob_ref, oc_ref)):
                        w1, b1, w2, b2 = hw[4 * j:4 * j + 4]
                        t = jnp.maximum(
                            jnp.dot(h, w1[...],
                                    preferred_element_type=jnp.float32)
                            + b1[...], 0.0)
                        o_ref[...] = (
                            jnp.dot(t, w2[...],
                                    preferred_element_type=jnp.float32)
                            + b2[...])

    def in3(ph, i):
        return (0, jnp.where(ph == 0, i, _NBLK - 1), 0)

    def in2(ph, i):
        return (jnp.where(ph == 0, i, _NBLK - 1), 0)

    def out2(ph, i):
        return (jnp.where(ph == 1, i, 0), 0)

    bcast = lambda ph, i: (0, 0)
    args = [p, degp, x, Wl, bl.reshape(1, _D), Wr,
            gw.reshape(1, _D), gb.reshape(1, _D), ga.reshape(1, _D)]
    in_specs = [
        pl.BlockSpec((2, _BR, _D), in3),
        pl.BlockSpec((2, _BR, _D), in3),
        pl.BlockSpec((_BR, _D), in2),
        pl.BlockSpec((_D, _D), bcast),
        pl.BlockSpec((1, _D), bcast),
        pl.BlockSpec((_D, _D), bcast),
        pl.BlockSpec((1, _D), bcast),
        pl.BlockSpec((1, _D), bcast),
        pl.BlockSpec((1, _D), bcast),
    ]
    oblk = pl.BlockSpec((_BR, _D), out2)
    osh = jax.ShapeDtypeStruct((_N, _D), jnp.float32)
    out_specs = [oblk]
    out_shape = [osh]
    if with_heads:
        for (w1, b1, w2, b2) in heads_params:
            args += [w1, b1.reshape(1, _MLP),
                     jnp.pad(w2, ((0, 0), (0, _D - w2.shape[1]))),
                     jnp.pad(b2, (0, _D - b2.shape[0])).reshape(1, _D)]
            in_specs += [pl.BlockSpec((_D, _MLP), bcast),
                         pl.BlockSpec((1, _MLP), bcast),
                         pl.BlockSpec((_MLP, _D), bcast),
                         pl.BlockSpec((1, _D), bcast)]
        out_specs += [oblk, oblk, oblk]
        out_shape += [osh, osh, osh]

    return pl.pallas_call(
        body,
        grid=(2, _NBLK),
        in_specs=in_specs,
        out_specs=out_specs,
        out_shape=out_shape,
        scratch_shapes=[pltpu.VMEM((_N, _D), jnp.float32),
                        pltpu.VMEM((1, _D), jnp.float32),
                        pltpu.VMEM((1, _D), jnp.float32)],
    )(*args)


def kernel(x, edge_index, Wl1, bl1, Wr1, Wl2, bl2, Wr2, Wl3, bl3, Wr3,
           gn1_w, gn1_b, gn1_a, gn2_w, gn2_b, gn2_a, gn3_w, gn3_b, gn3_a,
           hh_W1, hh_b1, hh_W2, hh_b2, eth_W1, eth_b1, eth_W2, eth_b2,
           rel_W1, rel_b1, rel_W2, rel_b2):
    src = edge_index[0].reshape(_NW, _NG, _G, _C)
    dst = edge_index[1].reshape(_NW, _NG, _G, _C)

    (degp,) = _sc_deg(x, src, dst)
    (p1,) = _sc_agg(x, src, dst)
    (h1,) = _tc_layer(p1, degp, x, Wl1, bl1, Wr1, gn1_w, gn1_b, gn1_a)

    (p2,) = _sc_agg(h1, src, dst)
    (h2,) = _tc_layer(p2, degp, h1, Wl2, bl2, Wr2, gn2_w, gn2_b, gn2_a)

    (p3,) = _sc_agg(h2, src, dst)
    _, oa, ob, oc = _tc_layer(
        p3, degp, h2, Wl3, bl3, Wr3, gn3_w, gn3_b, gn3_a,
        heads_params=((hh_W1, hh_b1, hh_W2, hh_b2),
                      (eth_W1, eth_b1, eth_W2, eth_b2),
                      (rel_W1, rel_b1, rel_W2, rel_b2)))
    return (oa[:_NUM_HH, :_O_HH], ob[:_NUM_HH, :_O_ETH], oc[:_NUM_HH, :_O_REL])
